# Initial kernel scaffold; baseline (speedup 1.0000x reference)
import jax
import jax.numpy as jnp
from jax.experimental import pallas as pl


def kernel(node_ids, edge_index, etype, entity, W1, W2):
    raise NotImplementedError("write your pallas kernel here")


# speedup vs baseline: 7.9798x; 7.9798x over previous
import jax
import jax.numpy as jnp
from jax import lax
from jax.experimental import pallas as pl
from jax.experimental.pallas import tpu as pltpu
from jax.experimental.pallas import tpu_sc as plsc

N = 10000
E = 320000
D = 128
R = 8
T = 64

NC = 2
NS = 16
NW = NC * NS
LANES = 16
EW = E // NW
CHUNK = 80
SUB = CHUNK // LANES
NCHUNKS = EW // CHUNK
CW = 16
CROWS = 640
NP = 10240
NZS = 10
SROWS = N // NZS

BN = 2000
BNP = 2048
HPW = NP // NW
HCH = 80
HNCH = HPW // HCH


def _make_edge_agg(dout: int, with_cnt: bool):
  mesh = plsc.VectorSubcoreMesh(core_axis_name="c", subcore_axis_name="s",
                                num_cores=NC, num_subcores=NS)
  out_type = [jax.ShapeDtypeStruct((NC, NP, dout), jnp.float32)]
  if with_cnt:
    out_type.append(jax.ShapeDtypeStruct((NC, CROWS, CW), jnp.float32))
  scratch = [
      pltpu.VMEM((NCHUNKS, CHUNK), jnp.int32),
      pltpu.VMEM((NCHUNKS, CHUNK), jnp.int32),
      pltpu.VMEM((LANES, dout), jnp.float32),
      pltpu.VMEM_SHARED((N, dout), jnp.float32),
      pltpu.SemaphoreType.DMA,
  ]
  if with_cnt:
    scratch.append(pltpu.VMEM((LANES, CW), jnp.float32))
    scratch.append(pltpu.VMEM_SHARED((CROWS, CW), jnp.float32))

  def body(*refs):
    it = iter(refs)
    table_r = next(it)
    idx_r = next(it)
    dst_r = next(it)
    zeros_r = next(it)
    zcnt_r = next(it) if with_cnt else None
    oh_r = next(it) if with_cnt else None
    part_r = next(it)
    cnt_r = next(it) if with_cnt else None
    idx_v = next(it)
    dst_v = next(it)
    buf_v = next(it)
    acc_sh = next(it)
    sem = next(it)
    if with_cnt:
      ohbuf_v = next(it)
      cnt_sh = next(it)

    cid = lax.axis_index("c")
    sid = lax.axis_index("s")
    wid = sid * NC + cid

    pltpu.sync_copy(idx_r.at[wid], idx_v)
    pltpu.sync_copy(dst_r.at[wid], dst_v)

    @pl.when(sid < NZS)
    def _zero_acc():
      stripe = pl.ds(sid * SROWS, SROWS)
      pltpu.sync_copy(zeros_r.at[stripe], acc_sh.at[stripe])

    if with_cnt:
      @pl.when(sid == NZS)
      def _zero_cnt():
        pltpu.sync_copy(zcnt_r, cnt_sh)

    plsc.subcore_barrier()

    def chunk_body(j, c):
      for m in range(SUB):
        ivec = idx_v[j, pl.ds(m * LANES, LANES)]
        dvec = dst_v[j, pl.ds(m * LANES, LANES)]
        pltpu.async_copy(table_r.at[ivec], buf_v, sem).wait()
        pltpu.sync_copy(buf_v, acc_sh.at[dvec], add=True)
        if with_cnt:
          cvec = jnp.bitwise_and(dvec, LANES - 1)
          rvec = lax.shift_right_logical(dvec, 4)
          pltpu.async_copy(oh_r.at[cvec], ohbuf_v, sem).wait()
          pltpu.sync_copy(ohbuf_v, cnt_sh.at[rvec], add=True)
      return c

    lax.fori_loop(0, NCHUNKS, chunk_body, 0)
    plsc.subcore_barrier()

    @pl.when(sid < NZS)
    def _drain():
      stripe = pl.ds(sid * SROWS, SROWS)
      pltpu.sync_copy(acc_sh.at[stripe], part_r.at[cid, stripe])

    if with_cnt:
      @pl.when(sid == NZS)
      def _drain_cnt():
        pltpu.sync_copy(cnt_sh, cnt_r.at[cid])

  params = pltpu.CompilerParams(use_tc_tiling_on_sc=False)
  return pl.kernel(body, out_type=out_type, mesh=mesh, scratch_types=scratch,
                   compiler_params=params)


_edge_agg_l1 = _make_edge_agg(D, with_cnt=True)
_edge_agg_l2 = _make_edge_agg(T, with_cnt=False)


def _gather_h_body(ent_r, nid_r, h_r, idx_v, buf_v, sem):
  cid = lax.axis_index("c")
  sid = lax.axis_index("s")
  wid = sid * NC + cid
  pltpu.sync_copy(nid_r.at[wid], idx_v)
  for q in range(HNCH):
    pltpu.async_copy(ent_r.at[idx_v.at[q]], buf_v, sem).wait()
    pltpu.sync_copy(buf_v, h_r.at[pl.ds(wid * HPW + q * HCH, HCH)])


_gather_h = pl.kernel(
    _gather_h_body,
    out_type=[jax.ShapeDtypeStruct((NP, D), jnp.float32)],
    mesh=plsc.VectorSubcoreMesh(core_axis_name="c", subcore_axis_name="s",
                                num_cores=NC, num_subcores=NS),
    scratch_types=[
        pltpu.VMEM((HNCH, HCH), jnp.int32),
        pltpu.VMEM((HCH, D), jnp.float32),
        pltpu.SemaphoreType.DMA,
    ],
)


def _edge_idx_body(s_ref, e_ref, o_ref):
  o_ref[...] = e_ref[...] * NP + s_ref[...]


_edge_idx = pl.pallas_call(
    _edge_idx_body,
    grid=(NW,),
    in_specs=[
        pl.BlockSpec((1, NCHUNKS, CHUNK), lambda i: (i, 0, 0)),
        pl.BlockSpec((1, NCHUNKS, CHUNK), lambda i: (i, 0, 0)),
    ],
    out_specs=pl.BlockSpec((1, NCHUNKS, CHUNK), lambda i: (i, 0, 0)),
    out_shape=jax.ShapeDtypeStruct((NW, NCHUNKS, CHUNK), jnp.int32),
)


def _table1_body(h_ref, w_ref, o_ref):
  o_ref[0] = jnp.dot(h_ref[...], w_ref[0], preferred_element_type=jnp.float32)


_table1 = pl.pallas_call(
    _table1_body,
    grid=(R, NP // BNP),
    in_specs=[
        pl.BlockSpec((BNP, D), lambda r, i: (i, 0)),
        pl.BlockSpec((1, D, D), lambda r, i: (r, 0, 0)),
    ],
    out_specs=pl.BlockSpec((1, BNP, D), lambda r, i: (r, i, 0)),
    out_shape=jax.ShapeDtypeStruct((R, NP, D), jnp.float32),
)


def _layer2_body(p_ref, c_ref, w_ref, t_ref, inv_ref):
  cnt = c_ref[0] + c_ref[1]
  inv = 1.0 / jnp.maximum(cnt, 1.0)
  h1 = jnp.maximum((p_ref[0] + p_ref[1]) * inv, 0.0)
  t_ref[0] = jnp.dot(h1, w_ref[0], preferred_element_type=jnp.float32)
  inv_ref[...] = inv


_layer2 = pl.pallas_call(
    _layer2_body,
    grid=(R, NP // BNP),
    in_specs=[
        pl.BlockSpec((NC, BNP, D), lambda r, i: (0, i, 0)),
        pl.BlockSpec((NC, BNP, 1), lambda r, i: (0, i, 0)),
        pl.BlockSpec((1, D, T), lambda r, i: (r, 0, 0)),
    ],
    out_specs=[
        pl.BlockSpec((1, BNP, T), lambda r, i: (r, i, 0)),
        pl.BlockSpec((BNP, 1), lambda r, i: (i, 0)),
    ],
    out_shape=[
        jax.ShapeDtypeStruct((R, NP, T), jnp.float32),
        jax.ShapeDtypeStruct((NP, 1), jnp.float32),
    ],
)


def _final_body(p_ref, inv_ref, o_ref):
  o_ref[...] = (p_ref[0] + p_ref[1]) * inv_ref[...]


_final = pl.pallas_call(
    _final_body,
    grid=(N // BN,),
    in_specs=[
        pl.BlockSpec((NC, BN, T), lambda i: (0, i, 0)),
        pl.BlockSpec((BN, 1), lambda i: (i, 0)),
    ],
    out_specs=pl.BlockSpec((BN, T), lambda i: (i, 0)),
    out_shape=jax.ShapeDtypeStruct((N, T), jnp.float32),
)


def kernel(node_ids, edge_index, etype, entity, W1, W2):
  node_ids = node_ids.astype(jnp.int32)
  src = edge_index[0].astype(jnp.int32).reshape(NW, NCHUNKS, CHUNK)
  dst = edge_index[1].astype(jnp.int32).reshape(NW, NCHUNKS, CHUNK)
  et = etype.astype(jnp.int32).reshape(NW, NCHUNKS, CHUNK)
  entity = entity.astype(jnp.float32)

  zeros_d = jnp.zeros((N, D), jnp.float32)
  zeros_t = jnp.zeros((N, T), jnp.float32)
  zcnt = jnp.zeros((CROWS, CW), jnp.float32)
  onehot = jnp.eye(CW, dtype=jnp.float32)

  nid_pad = jnp.concatenate(
      [node_ids, jnp.zeros((NP - N,), jnp.int32)]).reshape(NW, HNCH, HCH)
  [h] = _gather_h(entity, nid_pad)
  idx3 = _edge_idx(src, et)
  table1 = _table1(h, W1.astype(jnp.float32)).reshape(R * NP, D)
  part1, cnt = _edge_agg_l1(table1, idx3, dst, zeros_d, zcnt, onehot)
  cntn = cnt.reshape(NC, NP, 1)
  table2, inv = _layer2(part1, cntn, W2.astype(jnp.float32))
  [part2] = _edge_agg_l2(table2.reshape(R * NP, T), idx3, dst, zeros_t)
  out = _final(part2, inv)
  return out[None, :, :], jnp.ones((1, N), out.dtype)


# speedup vs baseline: 8.8477x; 1.1088x over previous
import jax
import jax.numpy as jnp
from jax import lax
from jax.experimental import pallas as pl
from jax.experimental.pallas import tpu as pltpu
from jax.experimental.pallas import tpu_sc as plsc

N = 10000
E = 320000
D = 128
R = 8
T = 64

NC = 2
NS = 16
NW = NC * NS
LANES = 16
EW = E // NW
CHUNK = 80
SUB = CHUNK // LANES
NCHUNKS = EW // CHUNK
CW = 16
CROWS = 640
NP = 10240
NZS = 10
SROWS = N // NZS

BN = 2000
BNP = 2048
HPW = NP // NW
HCH = 80
HNCH = HPW // HCH


def _make_edge_agg(dout: int, with_cnt: bool):
  mesh = plsc.VectorSubcoreMesh(core_axis_name="c", subcore_axis_name="s",
                                num_cores=NC, num_subcores=NS)
  out_type = [jax.ShapeDtypeStruct((NC, NP, dout), jnp.float32)]
  if with_cnt:
    out_type.append(jax.ShapeDtypeStruct((NC, CROWS, CW), jnp.float32))
  scratch = [
      pltpu.VMEM((NCHUNKS, CHUNK), jnp.int32),
      pltpu.VMEM((NCHUNKS, CHUNK), jnp.int32),
      pltpu.VMEM((LANES, dout), jnp.float32),
      pltpu.VMEM((LANES, dout), jnp.float32),
      pltpu.VMEM_SHARED((N, dout), jnp.float32),
      pltpu.SemaphoreType.DMA,
  ]
  if with_cnt:
    scratch.append(pltpu.VMEM((LANES, CW), jnp.float32))
    scratch.append(pltpu.VMEM((LANES, CW), jnp.float32))
    scratch.append(pltpu.SemaphoreType.DMA)
    scratch.append(pltpu.VMEM_SHARED((CROWS, CW), jnp.float32))

  def body(*refs):
    it = iter(refs)
    table_r = next(it)
    idx_r = next(it)
    dst_r = next(it)
    zeros_r = next(it)
    zcnt_r = next(it) if with_cnt else None
    oh_r = next(it) if with_cnt else None
    part_r = next(it)
    cnt_r = next(it) if with_cnt else None
    idx_v = next(it)
    dst_v = next(it)
    bufs = [next(it), next(it)]
    acc_sh = next(it)
    sem = next(it)
    if with_cnt:
      ohbufs = [next(it), next(it)]
      sem2 = next(it)
      cnt_sh = next(it)

    cid = lax.axis_index("c")
    sid = lax.axis_index("s")
    wid = sid * NC + cid

    pltpu.sync_copy(idx_r.at[wid], idx_v)
    pltpu.sync_copy(dst_r.at[wid], dst_v)

    @pl.when(sid < NZS)
    def _zero_acc():
      stripe = pl.ds(sid * SROWS, SROWS)
      pltpu.sync_copy(zeros_r.at[stripe], acc_sh.at[stripe])

    if with_cnt:
      @pl.when(sid == NZS)
      def _zero_cnt():
        pltpu.sync_copy(zcnt_r, cnt_sh)

    plsc.subcore_barrier()

    def chunk_body(j, c):
      ivs = [idx_v[j, pl.ds(m * LANES, LANES)] for m in range(SUB)]
      dvs = [dst_v[j, pl.ds(m * LANES, LANES)] for m in range(SUB)]
      g = pltpu.async_copy(table_r.at[ivs[0]], bufs[0], sem)
      if with_cnt:
        cvs = [jnp.bitwise_and(dv, LANES - 1) for dv in dvs]
        rvs = [lax.shift_right_logical(dv, 4) for dv in dvs]
        og = pltpu.async_copy(oh_r.at[cvs[0]], ohbufs[0], sem2)
      for m in range(SUB):
        g.wait()
        if m + 1 < SUB:
          g = pltpu.async_copy(table_r.at[ivs[m + 1]], bufs[(m + 1) % 2], sem)
        pltpu.sync_copy(bufs[m % 2], acc_sh.at[dvs[m]], add=True)
        if with_cnt:
          og.wait()
          if m + 1 < SUB:
            og = pltpu.async_copy(oh_r.at[cvs[m + 1]], ohbufs[(m + 1) % 2], sem2)
          pltpu.sync_copy(ohbufs[m % 2], cnt_sh.at[rvs[m]], add=True)
      return c

    lax.fori_loop(0, NCHUNKS, chunk_body, 0)
    plsc.subcore_barrier()

    @pl.when(sid < NZS)
    def _drain():
      stripe = pl.ds(sid * SROWS, SROWS)
      pltpu.sync_copy(acc_sh.at[stripe], part_r.at[cid, stripe])

    if with_cnt:
      @pl.when(sid == NZS)
      def _drain_cnt():
        pltpu.sync_copy(cnt_sh, cnt_r.at[cid])

  params = pltpu.CompilerParams(use_tc_tiling_on_sc=False)
  return pl.kernel(body, out_type=out_type, mesh=mesh, scratch_types=scratch,
                   compiler_params=params)


_edge_agg_l1 = _make_edge_agg(D, with_cnt=True)
_edge_agg_l2 = _make_edge_agg(T, with_cnt=False)


def _gather_h_body(ent_r, nid_r, h_r, idx_v, buf_v, sem):
  cid = lax.axis_index("c")
  sid = lax.axis_index("s")
  wid = sid * NC + cid
  pltpu.sync_copy(nid_r.at[wid], idx_v)
  for q in range(HNCH):
    pltpu.async_copy(ent_r.at[idx_v.at[q]], buf_v, sem).wait()
    pltpu.sync_copy(buf_v, h_r.at[pl.ds(wid * HPW + q * HCH, HCH)])


_gather_h = pl.kernel(
    _gather_h_body,
    out_type=[jax.ShapeDtypeStruct((NP, D), jnp.float32)],
    mesh=plsc.VectorSubcoreMesh(core_axis_name="c", subcore_axis_name="s",
                                num_cores=NC, num_subcores=NS),
    scratch_types=[
        pltpu.VMEM((HNCH, HCH), jnp.int32),
        pltpu.VMEM((HCH, D), jnp.float32),
        pltpu.SemaphoreType.DMA,
    ],
)


def _edge_idx_body(s_ref, e_ref, o_ref):
  o_ref[...] = e_ref[...] * NP + s_ref[...]


_edge_idx = pl.pallas_call(
    _edge_idx_body,
    grid=(NW,),
    in_specs=[
        pl.BlockSpec((1, NCHUNKS, CHUNK), lambda i: (i, 0, 0)),
        pl.BlockSpec((1, NCHUNKS, CHUNK), lambda i: (i, 0, 0)),
    ],
    out_specs=pl.BlockSpec((1, NCHUNKS, CHUNK), lambda i: (i, 0, 0)),
    out_shape=jax.ShapeDtypeStruct((NW, NCHUNKS, CHUNK), jnp.int32),
)


def _table1_body(h_ref, w_ref, o_ref):
  o_ref[0] = jnp.dot(h_ref[...], w_ref[0], preferred_element_type=jnp.float32)


_table1 = pl.pallas_call(
    _table1_body,
    grid=(R, NP // BNP),
    in_specs=[
        pl.BlockSpec((BNP, D), lambda r, i: (i, 0)),
        pl.BlockSpec((1, D, D), lambda r, i: (r, 0, 0)),
    ],
    out_specs=pl.BlockSpec((1, BNP, D), lambda r, i: (r, i, 0)),
    out_shape=jax.ShapeDtypeStruct((R, NP, D), jnp.float32),
)


def _layer2_body(p_ref, c_ref, w_ref, t_ref, inv_ref):
  cnt = c_ref[0] + c_ref[1]
  inv = 1.0 / jnp.maximum(cnt, 1.0)
  h1 = jnp.maximum((p_ref[0] + p_ref[1]) * inv, 0.0)
  t_ref[0] = jnp.dot(h1, w_ref[0], preferred_element_type=jnp.float32)
  inv_ref[...] = inv


_layer2 = pl.pallas_call(
    _layer2_body,
    grid=(R, NP // BNP),
    in_specs=[
        pl.BlockSpec((NC, BNP, D), lambda r, i: (0, i, 0)),
        pl.BlockSpec((NC, BNP, 1), lambda r, i: (0, i, 0)),
        pl.BlockSpec((1, D, T), lambda r, i: (r, 0, 0)),
    ],
    out_specs=[
        pl.BlockSpec((1, BNP, T), lambda r, i: (r, i, 0)),
        pl.BlockSpec((BNP, 1), lambda r, i: (i, 0)),
    ],
    out_shape=[
        jax.ShapeDtypeStruct((R, NP, T), jnp.float32),
        jax.ShapeDtypeStruct((NP, 1), jnp.float32),
    ],
)


def _final_body(p_ref, inv_ref, o_ref):
  o_ref[...] = (p_ref[0] + p_ref[1]) * inv_ref[...]


_final = pl.pallas_call(
    _final_body,
    grid=(N // BN,),
    in_specs=[
        pl.BlockSpec((NC, BN, T), lambda i: (0, i, 0)),
        pl.BlockSpec((BN, 1), lambda i: (i, 0)),
    ],
    out_specs=pl.BlockSpec((BN, T), lambda i: (i, 0)),
    out_shape=jax.ShapeDtypeStruct((N, T), jnp.float32),
)


def kernel(node_ids, edge_index, etype, entity, W1, W2):
  node_ids = node_ids.astype(jnp.int32)
  src = edge_index[0].astype(jnp.int32).reshape(NW, NCHUNKS, CHUNK)
  dst = edge_index[1].astype(jnp.int32).reshape(NW, NCHUNKS, CHUNK)
  et = etype.astype(jnp.int32).reshape(NW, NCHUNKS, CHUNK)
  entity = entity.astype(jnp.float32)

  zeros_d = jnp.zeros((N, D), jnp.float32)
  zeros_t = jnp.zeros((N, T), jnp.float32)
  zcnt = jnp.zeros((CROWS, CW), jnp.float32)
  onehot = jnp.eye(CW, dtype=jnp.float32)

  nid_pad = jnp.concatenate(
      [node_ids, jnp.zeros((NP - N,), jnp.int32)]).reshape(NW, HNCH, HCH)
  [h] = _gather_h(entity, nid_pad)
  idx3 = _edge_idx(src, et)
  table1 = _table1(h, W1.astype(jnp.float32)).reshape(R * NP, D)
  part1, cnt = _edge_agg_l1(table1, idx3, dst, zeros_d, zcnt, onehot)
  cntn = cnt.reshape(NC, NP, 1)
  table2, inv = _layer2(part1, cntn, W2.astype(jnp.float32))
  [part2] = _edge_agg_l2(table2.reshape(R * NP, T), idx3, dst, zeros_t)
  out = _final(part2, inv)
  return out[None, :, :], jnp.ones((1, N), out.dtype)
